# Initial kernel scaffold; baseline (speedup 1.0000x reference)
import jax
import jax.numpy as jnp
from jax.experimental import pallas as pl


def kernel(x, edge_index, fc0_w, fc0_b, conv_w1, conv_w2, W_w, W_b, fcout_w, fcout_b, outlin_w, outlin_b):
    raise NotImplementedError("write your pallas kernel here")


# speedup vs baseline: 18.6686x; 18.6686x over previous
import functools

import jax
import jax.numpy as jnp
from jax import lax
from jax.experimental import pallas as pl
from jax.experimental.pallas import tpu as pltpu
from jax.experimental.pallas import tpu_sc as plsc

N = 10000
E = 320000
D = 128
H = 128
C = 64

NC = 2
NS = 16
NW = NC * NS
EPW = E // NW
CH = 80
NCH = EPW // CH
RPT = N // NS
DW = 16

_sc_mesh = plsc.VectorSubcoreMesh(core_axis_name="c", subcore_axis_name="s")
_sc_params = pltpu.CompilerParams(use_tc_tiling_on_sc=False)


@functools.partial(
    pl.kernel,
    out_type=jax.ShapeDtypeStruct((NC, N, DW), jnp.float32),
    mesh=_sc_mesh,
    scratch_types=[
        pltpu.VMEM((NCH, CH), jnp.int32),
        pltpu.VMEM((CH, DW), jnp.float32),
        pltpu.VMEM_SHARED((N, DW), jnp.float32),
    ],
    compiler_params=_sc_params,
)
def _sc_degree(dst_hbm, zeros_hbm, ones_hbm, out_hbm, didx, ones, acc):
    cid = lax.axis_index("c")
    sid = lax.axis_index("s")
    pltpu.sync_copy(dst_hbm.at[cid, sid], didx)
    pltpu.sync_copy(ones_hbm, ones)
    pltpu.sync_copy(zeros_hbm.at[pl.ds(sid * RPT, RPT)],
                    acc.at[pl.ds(sid * RPT, RPT)])
    plsc.subcore_barrier()

    def body(j, carry):
        pltpu.sync_copy(ones, acc.at[didx.at[j]], add=True)
        return carry

    lax.fori_loop(0, NCH, body, 0)
    plsc.subcore_barrier()
    pltpu.sync_copy(acc.at[pl.ds(sid * RPT, RPT)],
                    out_hbm.at[cid, pl.ds(sid * RPT, RPT)])


@functools.partial(
    pl.kernel,
    out_type=jax.ShapeDtypeStruct((NC, N, H), jnp.float32),
    mesh=_sc_mesh,
    scratch_types=[
        pltpu.VMEM((NCH, CH), jnp.int32),
        pltpu.VMEM((NCH, CH), jnp.int32),
        pltpu.VMEM((CH, H), jnp.float32),
        pltpu.VMEM_SHARED((N, H), jnp.float32),
        pltpu.SemaphoreType.DMA,
    ],
    compiler_params=_sc_params,
)
def _sc_edge_agg(src_hbm, dst_hbm, hs_hbm, zeros_hbm, out_hbm,
                 sidx, didx, rows, acc, sem):
    cid = lax.axis_index("c")
    sid = lax.axis_index("s")
    pltpu.sync_copy(src_hbm.at[cid, sid], sidx)
    pltpu.sync_copy(dst_hbm.at[cid, sid], didx)
    pltpu.sync_copy(zeros_hbm.at[pl.ds(sid * RPT, RPT)],
                    acc.at[pl.ds(sid * RPT, RPT)])
    plsc.subcore_barrier()

    def body(j, carry):
        pltpu.async_copy(hs_hbm.at[sidx.at[j]], rows, sem).wait()
        pltpu.sync_copy(rows, acc.at[didx.at[j]], add=True)
        return carry

    lax.fori_loop(0, NCH, body, 0)
    plsc.subcore_barrier()
    pltpu.sync_copy(acc.at[pl.ds(sid * RPT, RPT)],
                    out_hbm.at[cid, pl.ds(sid * RPT, RPT)])


_R = 1000


def _dinv(dga, dgb):
    return lax.rsqrt(dga[:, :1] + dgb[:, :1] + 1.0)


def _tc_inmid_body(x, f0t, f0b, c1t, dga, dgb, h0_o, hs1_o):
    h0 = jnp.maximum(x[...] @ f0t[...] + f0b[...], 0.0)
    h0_o[...] = h0
    hs1_o[...] = _dinv(dga[...], dgb[...]) * (h0 @ c1t[...])


def _tc_postmid_body(pa, pb, hs1, dga, dgb, wwt, wb, c2t, h1_o, hs2_o):
    dinv = _dinv(dga[...], dgb[...])
    agg = dinv * (pa[...] + pb[...] + hs1[...])
    h1 = jnp.maximum(agg @ wwt[...] + wb[...], 0.0)
    h1_o[...] = h1
    hs2_o[...] = dinv * (h1 @ c2t[...])


def _tc_postout_body(pa, pb, hs2, dga, dgb, wwt, wb, h0, h1,
                     fot, fob, olt, olb, out_o):
    dinv = _dinv(dga[...], dgb[...])
    agg = dinv * (pa[...] + pb[...] + hs2[...])
    h2 = jnp.maximum(agg @ wwt[...] + wb[...], 0.0)
    jk = jnp.maximum(jnp.maximum(h0[...], h1[...]), h2)
    t = jk @ fot[...] + fob[...]
    out_o[...] = t @ olt[...] + olb[...]


def _row_spec(w):
    return pl.BlockSpec((_R, w), lambda i: (i, 0))


def _whole_spec(r, w):
    return pl.BlockSpec((r, w), lambda i: (0, 0))


def _tc_call(body, in_specs, out_specs, out_shape, args):
    return pl.pallas_call(
        body,
        grid=(N // _R,),
        in_specs=in_specs,
        out_specs=out_specs,
        out_shape=out_shape,
    )(*args)


def kernel(x, edge_index, fc0_w, fc0_b, conv_w1, conv_w2, W_w, W_b,
           fcout_w, fcout_b, outlin_w, outlin_b):
    src = edge_index[0].reshape(NC, NS, NCH, CH)
    dst = edge_index[1].reshape(NC, NS, NCH, CH)
    zeros_h = jnp.zeros((N, H), jnp.float32)
    zeros_d = jnp.zeros((N, DW), jnp.float32)
    ones_d = jnp.ones((CH, DW), jnp.float32)

    degp = _sc_degree(dst, zeros_d, ones_d)
    dga, dgb = degp[0], degp[1]

    f32 = jnp.float32
    h0, hs1 = _tc_call(
        _tc_inmid_body,
        [_row_spec(D), _whole_spec(D, H), _whole_spec(1, H),
         _whole_spec(H, H), _row_spec(DW), _row_spec(DW)],
        [_row_spec(H), _row_spec(H)],
        (jax.ShapeDtypeStruct((N, H), f32), jax.ShapeDtypeStruct((N, H), f32)),
        (x, fc0_w.T, fc0_b.reshape(1, H), conv_w1.T, dga, dgb),
    )

    p1 = _sc_edge_agg(src, dst, hs1, zeros_h)
    h1, hs2 = _tc_call(
        _tc_postmid_body,
        [_row_spec(H), _row_spec(H), _row_spec(H), _row_spec(DW),
         _row_spec(DW), _whole_spec(H, H), _whole_spec(1, H),
         _whole_spec(H, H)],
        [_row_spec(H), _row_spec(H)],
        (jax.ShapeDtypeStruct((N, H), f32), jax.ShapeDtypeStruct((N, H), f32)),
        (p1[0], p1[1], hs1, dga, dgb, W_w.T, W_b.reshape(1, H), conv_w2.T),
    )

    p2 = _sc_edge_agg(src, dst, hs2, zeros_h)
    out = _tc_call(
        _tc_postout_body,
        [_row_spec(H), _row_spec(H), _row_spec(H), _row_spec(DW),
         _row_spec(DW), _whole_spec(H, H), _whole_spec(1, H),
         _row_spec(H), _row_spec(H), _whole_spec(H, H), _whole_spec(1, H),
         _whole_spec(H, C), _whole_spec(1, C)],
        _row_spec(C),
        jax.ShapeDtypeStruct((N, C), f32),
        (p2[0], p2[1], hs2, dga, dgb, W_w.T, W_b.reshape(1, H), h0, h1,
         fcout_w.T, fcout_b.reshape(1, H), outlin_w.T, outlin_b.reshape(1, C)),
    )
    return out
